# baseline (speedup 1.0000x reference)
import functools

import jax
import jax.numpy as jnp
from jax import lax
from jax.experimental import pallas as pl
from jax.experimental.pallas import tpu as pltpu
from jax.experimental.pallas import tpu_sc as plsc

N = 10000
DEG = 32
E = N * DEG
D = 128
NLAYERS = 4

SPLITS = 2
NH = N // SPLITS
SEG = E // SPLITS

NW = 32
ROWS_W = SEG // NW
CH = 100
NCH = ROWS_W // CH
K = 2
NG = NCH // K

NB = 5
BN = NH // NB


@functools.cache
def _sc_gather_fn():
    mesh = plsc.VectorSubcoreMesh(core_axis_name="c", subcore_axis_name="s")
    return functools.partial(
        pl.kernel,
        mesh=mesh,
        out_type=jax.ShapeDtypeStruct((SEG, D), jnp.float32),
        scratch_types=[
            pltpu.VMEM((NCH, CH), jnp.int32),
            pltpu.VMEM((2, K * CH, D), jnp.float32),
            pltpu.SemaphoreType.DMA,
            pltpu.SemaphoreType.DMA,
        ],
    )(_sc_gather_body)


def _sc_gather(table, idx3):
    return _sc_gather_fn()(table, idx3)


def _sc_gather_body(table_hbm, idx_hbm, out_hbm, idx_v, rows_v, gsem, wsem):
    wid = lax.axis_index("s") * 2 + lax.axis_index("c")
    base = wid * ROWS_W
    pltpu.sync_copy(idx_hbm.at[wid], idx_v)

    def group(g, carry):
        p = lax.rem(g, 2)

        @pl.when(g >= 2)
        def _():
            pltpu.make_async_copy(
                rows_v.at[p], out_hbm.at[pl.ds(0, K * CH)], wsem
            ).wait()

        descs = [
            pltpu.async_copy(
                table_hbm.at[idx_v.at[g * K + b]],
                rows_v.at[p, pl.ds(b * CH, CH)],
                gsem,
            )
            for b in range(K)
        ]
        for d in descs:
            d.wait()
        pltpu.async_copy(
            rows_v.at[p], out_hbm.at[pl.ds(base + g * (K * CH), K * CH)], wsem
        )
        return carry

    lax.fori_loop(0, NG, group, 0)
    for _ in range(2):
        pltpu.make_async_copy(
            rows_v.at[0], out_hbm.at[pl.ds(0, K * CH)], wsem
        ).wait()


_DOT = (((1,), (0,)), ((), ()))


def _lstm_body(g_ref, x_ref, win_ref, b_ref, wlt_ref, wrt_ref, bl_ref,
               out_ref, h_s, c_s, *, relu, resid):
    t = pl.program_id(1)

    @pl.when(t == 0)
    def _():
        h_s[...] = jnp.zeros_like(h_s)
        c_s[...] = jnp.zeros_like(c_s)

    h_bf = h_s[...].astype(jnp.bfloat16)
    cat = jnp.concatenate([g_ref[...].astype(jnp.bfloat16), h_bf],
                          axis=1)
    z = lax.dot_general(cat, win_ref[...], _DOT,
                        preferred_element_type=jnp.float32) + b_ref[...]
    gi = 1.0 / (1.0 + jnp.exp(-z[:, :D]))
    gf = 1.0 / (1.0 + jnp.exp(-z[:, D:2 * D]))
    gg = jnp.tanh(z[:, 2 * D:3 * D])
    go = 1.0 / (1.0 + jnp.exp(-z[:, 3 * D:]))
    c = gf * c_s[...] + gi * gg
    h_s[...] = go * jnp.tanh(c)
    c_s[...] = c

    @pl.when(t == DEG - 1)
    def _():
        xb = x_ref[...]
        out = (lax.dot_general(h_s[...].astype(jnp.bfloat16), wlt_ref[...],
                               _DOT, preferred_element_type=jnp.float32)
               + lax.dot_general(xb.astype(jnp.bfloat16), wrt_ref[...],
                                 _DOT, preferred_element_type=jnp.float32)
               + bl_ref[...])
        if resid:
            out = out + xb
        if relu:
            out = jnp.maximum(out, 0.0)
        out_ref[...] = out


def _lstm_half(g, xin, win, bsum, wlt, wrt, blv, half, relu, resid):
    return pl.pallas_call(
        functools.partial(_lstm_body, relu=relu, resid=resid),
        grid=(NB, DEG),
        in_specs=[
            pl.BlockSpec((BN, D), lambda nb, t: (t * NB + nb, 0)),
            pl.BlockSpec((BN, D), lambda nb, t, h=half: (h * NB + nb, 0)),
            pl.BlockSpec((2 * D, 4 * D), lambda nb, t: (0, 0)),
            pl.BlockSpec((1, 4 * D), lambda nb, t: (0, 0)),
            pl.BlockSpec((D, D), lambda nb, t: (0, 0)),
            pl.BlockSpec((D, D), lambda nb, t: (0, 0)),
            pl.BlockSpec((1, D), lambda nb, t: (0, 0)),
        ],
        out_specs=pl.BlockSpec((BN, D), lambda nb, t: (nb, 0)),
        out_shape=jax.ShapeDtypeStruct((NH, D), jnp.float32),
        scratch_shapes=[pltpu.VMEM((BN, D), jnp.float32),
                        pltpu.VMEM((BN, D), jnp.float32)],
        compiler_params=pltpu.CompilerParams(
            dimension_semantics=("arbitrary", "arbitrary")),
    )(g, xin, win, bsum, wlt, wrt, blv)


def kernel(x, edge_index, Wih, Whh, bih, bhh, Wl, bl, Wr):
    src = edge_index[0]
    srcT = src.reshape(N, DEG).T
    idx3 = [srcT[:, s * NH:(s + 1) * NH].reshape(NW, NCH, CH)
            for s in range(SPLITS)]
    h = x
    for l in range(NLAYERS):
        win = jnp.concatenate([Wih[l].T, Whh[l].T],
                              axis=0).astype(jnp.bfloat16)
        bsum = (bih[l] + bhh[l]).reshape(1, 4 * D)
        wlt = Wl[l].T.astype(jnp.bfloat16)
        wrt = Wr[l].T.astype(jnp.bfloat16)
        blv = bl[l].reshape(1, D)
        gs = [_sc_gather(h, idx3[s]) for s in range(SPLITS)]
        outs = [_lstm_half(gs[s], h, win, bsum, wlt, wrt, blv, half=s,
                           relu=(l < 3), resid=(l in (1, 2)))
                for s in range(SPLITS)]
        h = jnp.concatenate(outs, axis=0)
    return h


# speedup vs baseline: 1.4523x; 1.4523x over previous
import functools

import jax
import jax.numpy as jnp
from jax import lax
from jax.experimental import pallas as pl
from jax.experimental.pallas import tpu as pltpu
from jax.experimental.pallas import tpu_sc as plsc

N = 10000
DEG = 32
E = N * DEG
D = 128
NLAYERS = 4

SPLITS = 2
NH = N // SPLITS
SEG = E // SPLITS

NW = 32
ROWS_W = SEG // NW
CH = 100
NCH = ROWS_W // CH
K = 2
NG = NCH // K

NB = 1
BN = NH // NB


@functools.cache
def _sc_gather_fn():
    mesh = plsc.VectorSubcoreMesh(core_axis_name="c", subcore_axis_name="s")
    return functools.partial(
        pl.kernel,
        mesh=mesh,
        out_type=jax.ShapeDtypeStruct((SEG, D), jnp.float32),
        scratch_types=[
            pltpu.VMEM((NCH, CH), jnp.int32),
            pltpu.VMEM((2, K * CH, D), jnp.float32),
            pltpu.SemaphoreType.DMA,
            pltpu.SemaphoreType.DMA,
        ],
    )(_sc_gather_body)


def _sc_gather(table, idx3):
    return _sc_gather_fn()(table, idx3)


def _sc_gather_body(table_hbm, idx_hbm, out_hbm, idx_v, rows_v, gsem, wsem):
    wid = lax.axis_index("s") * 2 + lax.axis_index("c")
    base = wid * ROWS_W
    pltpu.sync_copy(idx_hbm.at[wid], idx_v)

    def group(g, carry):
        p = lax.rem(g, 2)

        @pl.when(g >= 2)
        def _():
            pltpu.make_async_copy(
                rows_v.at[p], out_hbm.at[pl.ds(0, K * CH)], wsem
            ).wait()

        descs = [
            pltpu.async_copy(
                table_hbm.at[idx_v.at[g * K + b]],
                rows_v.at[p, pl.ds(b * CH, CH)],
                gsem,
            )
            for b in range(K)
        ]
        for d in descs:
            d.wait()
        pltpu.async_copy(
            rows_v.at[p], out_hbm.at[pl.ds(base + g * (K * CH), K * CH)], wsem
        )
        return carry

    lax.fori_loop(0, NG, group, 0)
    for _ in range(2):
        pltpu.make_async_copy(
            rows_v.at[0], out_hbm.at[pl.ds(0, K * CH)], wsem
        ).wait()


_DOT = (((1,), (0,)), ((), ()))


def _lstm_body(g_ref, x_ref, win_ref, b_ref, wlt_ref, wrt_ref, bl_ref,
               out_ref, h_s, c_s, *, relu, resid):
    t = pl.program_id(1)

    @pl.when(t == 0)
    def _():
        h_s[...] = jnp.zeros_like(h_s)
        c_s[...] = jnp.zeros_like(c_s)

    h_bf = h_s[...].astype(jnp.bfloat16)
    cat = jnp.concatenate([g_ref[...].astype(jnp.bfloat16), h_bf],
                          axis=1)
    z = lax.dot_general(cat, win_ref[...], _DOT,
                        preferred_element_type=jnp.float32) + b_ref[...]
    gi = 1.0 / (1.0 + jnp.exp(-z[:, :D]))
    gf = 1.0 / (1.0 + jnp.exp(-z[:, D:2 * D]))
    gg = jnp.tanh(z[:, 2 * D:3 * D])
    go = 1.0 / (1.0 + jnp.exp(-z[:, 3 * D:]))
    c = gf * c_s[...] + gi * gg
    h_s[...] = go * jnp.tanh(c)
    c_s[...] = c

    @pl.when(t == DEG - 1)
    def _():
        xb = x_ref[...]
        out = (lax.dot_general(h_s[...].astype(jnp.bfloat16), wlt_ref[...],
                               _DOT, preferred_element_type=jnp.float32)
               + lax.dot_general(xb.astype(jnp.bfloat16), wrt_ref[...],
                                 _DOT, preferred_element_type=jnp.float32)
               + bl_ref[...])
        if resid:
            out = out + xb
        if relu:
            out = jnp.maximum(out, 0.0)
        out_ref[...] = out


def _lstm_half(g, xin, win, bsum, wlt, wrt, blv, half, relu, resid):
    return pl.pallas_call(
        functools.partial(_lstm_body, relu=relu, resid=resid),
        grid=(NB, DEG),
        in_specs=[
            pl.BlockSpec((BN, D), lambda nb, t: (t * NB + nb, 0)),
            pl.BlockSpec((BN, D), lambda nb, t, h=half: (h * NB + nb, 0)),
            pl.BlockSpec((2 * D, 4 * D), lambda nb, t: (0, 0)),
            pl.BlockSpec((1, 4 * D), lambda nb, t: (0, 0)),
            pl.BlockSpec((D, D), lambda nb, t: (0, 0)),
            pl.BlockSpec((D, D), lambda nb, t: (0, 0)),
            pl.BlockSpec((1, D), lambda nb, t: (0, 0)),
        ],
        out_specs=pl.BlockSpec((BN, D), lambda nb, t: (nb, 0)),
        out_shape=jax.ShapeDtypeStruct((NH, D), jnp.float32),
        scratch_shapes=[pltpu.VMEM((BN, D), jnp.float32),
                        pltpu.VMEM((BN, D), jnp.float32)],
        compiler_params=pltpu.CompilerParams(
            dimension_semantics=("arbitrary", "arbitrary")),
    )(g, xin, win, bsum, wlt, wrt, blv)


def kernel(x, edge_index, Wih, Whh, bih, bhh, Wl, bl, Wr):
    src = edge_index[0]
    srcT = src.reshape(N, DEG).T
    idx3 = [srcT[:, s * NH:(s + 1) * NH].reshape(NW, NCH, CH)
            for s in range(SPLITS)]
    h = x
    for l in range(NLAYERS):
        win = jnp.concatenate([Wih[l].T, Whh[l].T],
                              axis=0).astype(jnp.bfloat16)
        bsum = (bih[l] + bhh[l]).reshape(1, 4 * D)
        wlt = Wl[l].T.astype(jnp.bfloat16)
        wrt = Wr[l].T.astype(jnp.bfloat16)
        blv = bl[l].reshape(1, D)
        gs = [_sc_gather(h, idx3[s]) for s in range(SPLITS)]
        outs = [_lstm_half(gs[s], h, win, bsum, wlt, wrt, blv, half=s,
                           relu=(l < 3), resid=(l in (1, 2)))
                for s in range(SPLITS)]
        h = jnp.concatenate(outs, axis=0)
    return h


# speedup vs baseline: 1.7020x; 1.1719x over previous
import functools

import jax
import jax.numpy as jnp
from jax import lax
from jax.experimental import pallas as pl
from jax.experimental.pallas import tpu as pltpu
from jax.experimental.pallas import tpu_sc as plsc

N = 10000
DEG = 32
E = N * DEG
D = 128
NLAYERS = 4

SPLITS = 2
NH = N // SPLITS
SEG = E // SPLITS

NW = 32
ROWS_W = SEG // NW
CH = 100
NCH = ROWS_W // CH
K = 2
NG = NCH // K

NB = 1
BN = NH // NB


@functools.cache
def _sc_gather_fn():
    mesh = plsc.VectorSubcoreMesh(core_axis_name="c", subcore_axis_name="s")
    return functools.partial(
        pl.kernel,
        mesh=mesh,
        out_type=jax.ShapeDtypeStruct((SEG, D), jnp.float32),
        scratch_types=[
            pltpu.VMEM((NCH, CH), jnp.int32),
            pltpu.VMEM((2, K * CH, D), jnp.float32),
            pltpu.SemaphoreType.DMA,
            pltpu.SemaphoreType.DMA,
        ],
    )(_sc_gather_body)


def _sc_gather(table, idx3):
    return _sc_gather_fn()(table, idx3)


def _sc_gather_body(table_hbm, idx_hbm, out_hbm, idx_v, rows_v, gsem, wsem):
    wid = lax.axis_index("s") * 2 + lax.axis_index("c")
    base = wid * ROWS_W
    pltpu.sync_copy(idx_hbm.at[wid], idx_v)

    def group(g, carry):
        p = lax.rem(g, 2)

        @pl.when(g >= 2)
        def _():
            pltpu.make_async_copy(
                rows_v.at[p], out_hbm.at[pl.ds(0, K * CH)], wsem
            ).wait()

        descs = [
            pltpu.async_copy(
                table_hbm.at[idx_v.at[g * K + b]],
                rows_v.at[p, pl.ds(b * CH, CH)],
                gsem,
            )
            for b in range(K)
        ]
        for d in descs:
            d.wait()
        pltpu.async_copy(
            rows_v.at[p], out_hbm.at[pl.ds(base + g * (K * CH), K * CH)], wsem
        )
        return carry

    lax.fori_loop(0, NG, group, 0)
    for _ in range(2):
        pltpu.make_async_copy(
            rows_v.at[0], out_hbm.at[pl.ds(0, K * CH)], wsem
        ).wait()


_DOT = (((1,), (0,)), ((), ()))


def _lstm_body(g_ref, x_ref, win_ref, b_ref, wlt_ref, wrt_ref, bl_ref,
               out_ref, h_s, c_s, *, relu, resid):
    t = pl.program_id(1)

    @pl.when(t == 0)
    def _():
        h_s[...] = jnp.zeros_like(h_s)
        c_s[...] = jnp.zeros_like(c_s)

    h_bf = h_s[...].astype(jnp.bfloat16)
    cat = jnp.concatenate([g_ref[...].astype(jnp.bfloat16), h_bf],
                          axis=1)
    z = lax.dot_general(cat, win_ref[...], _DOT,
                        preferred_element_type=jnp.float32) + b_ref[...]
    tt = jnp.tanh(z)
    gi = 0.5 * tt[:, :D] + 0.5
    gf = 0.5 * tt[:, D:2 * D] + 0.5
    gg = tt[:, 2 * D:3 * D]
    go = 0.5 * tt[:, 3 * D:] + 0.5
    c = gf * c_s[...] + gi * gg
    h_s[...] = go * jnp.tanh(c)
    c_s[...] = c

    @pl.when(t == DEG - 1)
    def _():
        xb = x_ref[...]
        out = (lax.dot_general(h_s[...].astype(jnp.bfloat16), wlt_ref[...],
                               _DOT, preferred_element_type=jnp.float32)
               + lax.dot_general(xb.astype(jnp.bfloat16), wrt_ref[...],
                                 _DOT, preferred_element_type=jnp.float32)
               + bl_ref[...])
        if resid:
            out = out + xb
        if relu:
            out = jnp.maximum(out, 0.0)
        out_ref[...] = out


def _lstm_half(g, xin, win, bsum, wlt, wrt, blv, half, relu, resid):
    return pl.pallas_call(
        functools.partial(_lstm_body, relu=relu, resid=resid),
        grid=(NB, DEG),
        in_specs=[
            pl.BlockSpec((BN, D), lambda nb, t: (t * NB + nb, 0)),
            pl.BlockSpec((BN, D), lambda nb, t, h=half: (h * NB + nb, 0)),
            pl.BlockSpec((2 * D, 4 * D), lambda nb, t: (0, 0)),
            pl.BlockSpec((1, 4 * D), lambda nb, t: (0, 0)),
            pl.BlockSpec((D, D), lambda nb, t: (0, 0)),
            pl.BlockSpec((D, D), lambda nb, t: (0, 0)),
            pl.BlockSpec((1, D), lambda nb, t: (0, 0)),
        ],
        out_specs=pl.BlockSpec((BN, D), lambda nb, t: (nb, 0)),
        out_shape=jax.ShapeDtypeStruct((NH, D), jnp.float32),
        scratch_shapes=[pltpu.VMEM((BN, D), jnp.float32),
                        pltpu.VMEM((BN, D), jnp.float32)],
        compiler_params=pltpu.CompilerParams(
            dimension_semantics=("arbitrary", "arbitrary")),
    )(g, xin, win, bsum, wlt, wrt, blv)


def kernel(x, edge_index, Wih, Whh, bih, bhh, Wl, bl, Wr):
    src = edge_index[0]
    srcT = src.reshape(N, DEG).T
    idx3 = [srcT[:, s * NH:(s + 1) * NH].reshape(NW, NCH, CH)
            for s in range(SPLITS)]
    h = x
    halfs = jnp.concatenate([jnp.full((D,), 0.5, jnp.float32),
                             jnp.full((D,), 0.5, jnp.float32),
                             jnp.ones((D,), jnp.float32),
                             jnp.full((D,), 0.5, jnp.float32)])
    for l in range(NLAYERS):
        win = (jnp.concatenate([Wih[l].T, Whh[l].T], axis=0)
               * halfs).astype(jnp.bfloat16)
        bsum = ((bih[l] + bhh[l]) * halfs).reshape(1, 4 * D)
        wlt = Wl[l].T.astype(jnp.bfloat16)
        wrt = Wr[l].T.astype(jnp.bfloat16)
        blv = bl[l].reshape(1, D)
        gs = [_sc_gather(h, idx3[s]) for s in range(SPLITS)]
        outs = [_lstm_half(gs[s], h, win, bsum, wlt, wrt, blv, half=s,
                           relu=(l < 3), resid=(l in (1, 2)))
                for s in range(SPLITS)]
        h = jnp.concatenate(outs, axis=0)
    return h


# speedup vs baseline: 1.7660x; 1.0376x over previous
import functools

import jax
import jax.numpy as jnp
from jax import lax
from jax.experimental import pallas as pl
from jax.experimental.pallas import tpu as pltpu
from jax.experimental.pallas import tpu_sc as plsc

N = 10000
DEG = 32
E = N * DEG
D = 128
NLAYERS = 4

SPLITS = 2
NH = N // SPLITS
SEG = E // SPLITS

NW = 32
ROWS_W = SEG // NW
CH = 100
NCH = ROWS_W // CH
K = 2
NG = NCH // K

NB = 1
BN = NH // NB


@functools.cache
def _sc_gather_fn():
    mesh = plsc.VectorSubcoreMesh(core_axis_name="c", subcore_axis_name="s")
    return functools.partial(
        pl.kernel,
        mesh=mesh,
        out_type=jax.ShapeDtypeStruct((SEG, D), jnp.float32),
        scratch_types=[
            pltpu.VMEM((NCH, CH), jnp.int32),
            pltpu.VMEM((2, K * CH, D), jnp.float32),
            pltpu.SemaphoreType.DMA,
            pltpu.SemaphoreType.DMA,
        ],
    )(_sc_gather_body)


def _sc_gather(table, idx3):
    return _sc_gather_fn()(table, idx3)


def _sc_gather_body(table_hbm, idx_hbm, out_hbm, idx_v, rows_v, gsem, wsem):
    wid = lax.axis_index("s") * 2 + lax.axis_index("c")
    base = wid * ROWS_W
    pltpu.sync_copy(idx_hbm.at[wid], idx_v)

    def group(g, carry):
        p = lax.rem(g, 2)

        @pl.when(g >= 2)
        def _():
            pltpu.make_async_copy(
                rows_v.at[p], out_hbm.at[pl.ds(0, K * CH)], wsem
            ).wait()

        descs = [
            pltpu.async_copy(
                table_hbm.at[idx_v.at[g * K + b]],
                rows_v.at[p, pl.ds(b * CH, CH)],
                gsem,
            )
            for b in range(K)
        ]
        for d in descs:
            d.wait()
        pltpu.async_copy(
            rows_v.at[p], out_hbm.at[pl.ds(base + g * (K * CH), K * CH)], wsem
        )
        return carry

    lax.fori_loop(0, NG, group, 0)
    for _ in range(2):
        pltpu.make_async_copy(
            rows_v.at[0], out_hbm.at[pl.ds(0, K * CH)], wsem
        ).wait()


_DOT = (((1,), (0,)), ((), ()))


def _lstm_body(g_ref, x_ref, win_ref, b_ref, wlt_ref, wrt_ref, bl_ref,
               out_ref, h_s, c_s, *, relu, resid):
    t = pl.program_id(1)

    @pl.when(t == 0)
    def _():
        h_s[...] = jnp.zeros_like(h_s)
        c_s[...] = jnp.zeros_like(c_s)

    h = h_s[...]
    c = c_s[...]
    for sub in range(2):
        cat = jnp.concatenate(
            [g_ref[sub * BN:(sub + 1) * BN].astype(jnp.bfloat16),
             h.astype(jnp.bfloat16)], axis=1)
        z = lax.dot_general(cat, win_ref[...], _DOT,
                            preferred_element_type=jnp.float32) + b_ref[...]
        tt = jnp.tanh(z)
        gi = 0.5 * tt[:, :D] + 0.5
        gf = 0.5 * tt[:, D:2 * D] + 0.5
        gg = tt[:, 2 * D:3 * D]
        go = 0.5 * tt[:, 3 * D:] + 0.5
        c = gf * c + gi * gg
        h = go * jnp.tanh(c)
    h_s[...] = h
    c_s[...] = c

    @pl.when(t == DEG // 2 - 1)
    def _():
        xb = x_ref[...]
        out = (lax.dot_general(h.astype(jnp.bfloat16), wlt_ref[...],
                               _DOT, preferred_element_type=jnp.float32)
               + lax.dot_general(xb.astype(jnp.bfloat16), wrt_ref[...],
                                 _DOT, preferred_element_type=jnp.float32)
               + bl_ref[...])
        if resid:
            out = out + xb
        if relu:
            out = jnp.maximum(out, 0.0)
        out_ref[...] = out


def _lstm_half(g, xin, win, bsum, wlt, wrt, blv, half, relu, resid):
    return pl.pallas_call(
        functools.partial(_lstm_body, relu=relu, resid=resid),
        grid=(NB, DEG // 2),
        in_specs=[
            pl.BlockSpec((2 * BN, D), lambda nb, t: (t * NB + nb, 0)),
            pl.BlockSpec((BN, D), lambda nb, t, h=half: (h * NB + nb, 0)),
            pl.BlockSpec((2 * D, 4 * D), lambda nb, t: (0, 0)),
            pl.BlockSpec((1, 4 * D), lambda nb, t: (0, 0)),
            pl.BlockSpec((D, D), lambda nb, t: (0, 0)),
            pl.BlockSpec((D, D), lambda nb, t: (0, 0)),
            pl.BlockSpec((1, D), lambda nb, t: (0, 0)),
        ],
        out_specs=pl.BlockSpec((BN, D), lambda nb, t: (nb, 0)),
        out_shape=jax.ShapeDtypeStruct((NH, D), jnp.float32),
        scratch_shapes=[pltpu.VMEM((BN, D), jnp.float32),
                        pltpu.VMEM((BN, D), jnp.float32)],
        compiler_params=pltpu.CompilerParams(
            dimension_semantics=("arbitrary", "arbitrary")),
    )(g, xin, win, bsum, wlt, wrt, blv)


def kernel(x, edge_index, Wih, Whh, bih, bhh, Wl, bl, Wr):
    src = edge_index[0]
    srcT = src.reshape(N, DEG).T
    idx3 = [srcT[:, s * NH:(s + 1) * NH].reshape(NW, NCH, CH)
            for s in range(SPLITS)]
    h = x
    halfs = jnp.concatenate([jnp.full((D,), 0.5, jnp.float32),
                             jnp.full((D,), 0.5, jnp.float32),
                             jnp.ones((D,), jnp.float32),
                             jnp.full((D,), 0.5, jnp.float32)])
    for l in range(NLAYERS):
        win = (jnp.concatenate([Wih[l].T, Whh[l].T], axis=0)
               * halfs).astype(jnp.bfloat16)
        bsum = ((bih[l] + bhh[l]) * halfs).reshape(1, 4 * D)
        wlt = Wl[l].T.astype(jnp.bfloat16)
        wrt = Wr[l].T.astype(jnp.bfloat16)
        blv = bl[l].reshape(1, D)
        gs = [_sc_gather(h, idx3[s]) for s in range(SPLITS)]
        outs = [_lstm_half(gs[s], h, win, bsum, wlt, wrt, blv, half=s,
                           relu=(l < 3), resid=(l in (1, 2)))
                for s in range(SPLITS)]
        h = jnp.concatenate(outs, axis=0)
    return h


# speedup vs baseline: 1.7691x; 1.0018x over previous
import functools

import jax
import jax.numpy as jnp
from jax import lax
from jax.experimental import pallas as pl
from jax.experimental.pallas import tpu as pltpu
from jax.experimental.pallas import tpu_sc as plsc

N = 10000
DEG = 32
E = N * DEG
D = 128
NLAYERS = 4

SPLITS = 2
NH = N // SPLITS
SEG = E // SPLITS

NW = 32
ROWS_W = SEG // NW
CH = 100
NCH = ROWS_W // CH
K = 2
NG = NCH // K

NB = 1
BN = NH // NB


@functools.cache
def _sc_gather_fn():
    mesh = plsc.VectorSubcoreMesh(core_axis_name="c", subcore_axis_name="s")
    return functools.partial(
        pl.kernel,
        mesh=mesh,
        out_type=jax.ShapeDtypeStruct((SEG, D), jnp.float32),
        scratch_types=[
            pltpu.VMEM((NCH, CH), jnp.int32),
            pltpu.VMEM((2, K * CH, D), jnp.float32),
            pltpu.SemaphoreType.DMA,
            pltpu.SemaphoreType.DMA,
        ],
    )(_sc_gather_body)


def _sc_gather(table, idx3):
    return _sc_gather_fn()(table, idx3)


def _sc_gather_body(table_hbm, idx_hbm, out_hbm, idx_v, rows_v, gsem, wsem):
    wid = lax.axis_index("s") * 2 + lax.axis_index("c")
    base = wid * ROWS_W
    pltpu.sync_copy(idx_hbm.at[wid], idx_v)

    def group(g, carry):
        p = lax.rem(g, 2)

        @pl.when(g >= 2)
        def _():
            pltpu.make_async_copy(
                rows_v.at[p], out_hbm.at[pl.ds(0, K * CH)], wsem
            ).wait()

        descs = [
            pltpu.async_copy(
                table_hbm.at[idx_v.at[g * K + b]],
                rows_v.at[p, pl.ds(b * CH, CH)],
                gsem,
            )
            for b in range(K)
        ]
        for d in descs:
            d.wait()
        pltpu.async_copy(
            rows_v.at[p], out_hbm.at[pl.ds(base + g * (K * CH), K * CH)], wsem
        )
        return carry

    lax.fori_loop(0, NG, group, 0)
    for _ in range(2):
        pltpu.make_async_copy(
            rows_v.at[0], out_hbm.at[pl.ds(0, K * CH)], wsem
        ).wait()


_DOT = (((1,), (0,)), ((), ()))


def _lstm_body(g_ref, x_ref, win_ref, b_ref, wlt_ref, wrt_ref, bl_ref,
               out_ref, h_s, c_s, *, relu, resid):
    t = pl.program_id(1)

    @pl.when(t == 0)
    def _():
        h_s[...] = jnp.zeros_like(h_s)
        c_s[...] = jnp.zeros_like(c_s)

    h = h_s[...]
    c = c_s[...]
    for sub in range(4):
        cat = jnp.concatenate(
            [g_ref[sub * BN:(sub + 1) * BN].astype(jnp.bfloat16),
             h.astype(jnp.bfloat16)], axis=1)
        z = lax.dot_general(cat, win_ref[...], _DOT,
                            preferred_element_type=jnp.float32) + b_ref[...]
        tt = jnp.tanh(z)
        gi = 0.5 * tt[:, :D] + 0.5
        gf = 0.5 * tt[:, D:2 * D] + 0.5
        gg = tt[:, 2 * D:3 * D]
        go = 0.5 * tt[:, 3 * D:] + 0.5
        c = gf * c + gi * gg
        h = go * jnp.tanh(c)
    h_s[...] = h
    c_s[...] = c

    @pl.when(t == DEG // 4 - 1)
    def _():
        xb = x_ref[...]
        out = (lax.dot_general(h.astype(jnp.bfloat16), wlt_ref[...],
                               _DOT, preferred_element_type=jnp.float32)
               + lax.dot_general(xb.astype(jnp.bfloat16), wrt_ref[...],
                                 _DOT, preferred_element_type=jnp.float32)
               + bl_ref[...])
        if resid:
            out = out + xb
        if relu:
            out = jnp.maximum(out, 0.0)
        out_ref[...] = out


def _lstm_half(g, xin, win, bsum, wlt, wrt, blv, half, relu, resid):
    return pl.pallas_call(
        functools.partial(_lstm_body, relu=relu, resid=resid),
        grid=(NB, DEG // 4),
        in_specs=[
            pl.BlockSpec((4 * BN, D), lambda nb, t: (t * NB + nb, 0)),
            pl.BlockSpec((BN, D), lambda nb, t, h=half: (h * NB + nb, 0)),
            pl.BlockSpec((2 * D, 4 * D), lambda nb, t: (0, 0)),
            pl.BlockSpec((1, 4 * D), lambda nb, t: (0, 0)),
            pl.BlockSpec((D, D), lambda nb, t: (0, 0)),
            pl.BlockSpec((D, D), lambda nb, t: (0, 0)),
            pl.BlockSpec((1, D), lambda nb, t: (0, 0)),
        ],
        out_specs=pl.BlockSpec((BN, D), lambda nb, t: (nb, 0)),
        out_shape=jax.ShapeDtypeStruct((NH, D), jnp.float32),
        scratch_shapes=[pltpu.VMEM((BN, D), jnp.float32),
                        pltpu.VMEM((BN, D), jnp.float32)],
        compiler_params=pltpu.CompilerParams(
            dimension_semantics=("arbitrary", "arbitrary")),
    )(g, xin, win, bsum, wlt, wrt, blv)


def kernel(x, edge_index, Wih, Whh, bih, bhh, Wl, bl, Wr):
    src = edge_index[0]
    srcT = src.reshape(N, DEG).T
    idx3 = [srcT[:, s * NH:(s + 1) * NH].reshape(NW, NCH, CH)
            for s in range(SPLITS)]
    h = x
    halfs = jnp.concatenate([jnp.full((D,), 0.5, jnp.float32),
                             jnp.full((D,), 0.5, jnp.float32),
                             jnp.ones((D,), jnp.float32),
                             jnp.full((D,), 0.5, jnp.float32)])
    for l in range(NLAYERS):
        win = (jnp.concatenate([Wih[l].T, Whh[l].T], axis=0)
               * halfs).astype(jnp.bfloat16)
        bsum = ((bih[l] + bhh[l]) * halfs).reshape(1, 4 * D)
        wlt = Wl[l].T.astype(jnp.bfloat16)
        wrt = Wr[l].T.astype(jnp.bfloat16)
        blv = bl[l].reshape(1, D)
        gs = [_sc_gather(h, idx3[s]) for s in range(SPLITS)]
        outs = [_lstm_half(gs[s], h, win, bsum, wlt, wrt, blv, half=s,
                           relu=(l < 3), resid=(l in (1, 2)))
                for s in range(SPLITS)]
        h = jnp.concatenate(outs, axis=0)
    return h
